# Initial kernel scaffold; baseline (speedup 1.0000x reference)
import jax
import jax.numpy as jnp
from jax.experimental import pallas as pl


def kernel(hidden_states, gate_weight):
    raise NotImplementedError("write your pallas kernel here")


# speedup vs baseline: 1.5287x; 1.5287x over previous
import functools

import jax
import jax.numpy as jnp
from jax.experimental import pallas as pl
from jax.experimental.pallas import tpu as pltpu

_TOP_K = 2
_ALPHA = 0.01
_TB = 512


def _router_body(x_ref, w_ref, wout_ref, iout_ref, stats_ref, *, n_tokens, n_experts):
    step = pl.program_id(0)
    nsteps = pl.num_programs(0)
    x = x_ref[...]
    w = w_ref[...]
    logits = jax.lax.dot_general(
        x, w, (((1,), (1,)), ((), ())), preferred_element_type=jnp.float32
    )

    lane = jax.lax.broadcasted_iota(jnp.int32, logits.shape, 1)
    m1 = jnp.max(logits, axis=-1, keepdims=True)
    i1 = jnp.min(jnp.where(logits == m1, lane, n_experts), axis=-1, keepdims=True)
    masked = jnp.where(lane == i1, -jnp.inf, logits)
    m2 = jnp.max(masked, axis=-1, keepdims=True)
    i2 = jnp.min(jnp.where(masked == m2, lane, n_experts), axis=-1, keepdims=True)

    ex = jnp.exp(logits - m1)
    z = jnp.sum(ex, axis=-1, keepdims=True)
    p1 = 1.0 / z
    p2 = jnp.exp(m2 - m1) / z
    denom = p1 + p2 + 1e-9
    wout_ref[...] = jnp.concatenate([p1 / denom, p2 / denom], axis=1)
    iout_ref[...] = jnp.concatenate([i1, i2], axis=1)

    probs_sum = jnp.sum(ex / z, axis=0, keepdims=True)
    one_hot = (lane == i1).astype(jnp.float32) + (lane == i2).astype(jnp.float32)
    cnt = jnp.sum(one_hot, axis=0, keepdims=True)

    @pl.when(step == 0)
    def _init():
        stats_ref[...] = jnp.zeros_like(stats_ref)

    stats_ref[0:1, :] += probs_sum
    stats_ref[1:2, :] += cnt

    @pl.when(step == nsteps - 1)
    def _finish():
        p_mean = stats_ref[0:1, :] / n_tokens
        f_mean = stats_ref[1:2, :] / (n_tokens * _TOP_K)
        aux = _ALPHA * n_experts * jnp.sum(p_mean * f_mean)
        stats_ref[2:3, :] = jnp.broadcast_to(aux, (1, n_experts))


def kernel(hidden_states, gate_weight):
    b, s, h = hidden_states.shape
    e = gate_weight.shape[0]
    t = b * s
    x = hidden_states.reshape(t, h)

    grid = (t // _TB,)
    body = functools.partial(_router_body, n_tokens=t, n_experts=e)
    wout, iout, stats = pl.pallas_call(
        body,
        grid=grid,
        in_specs=[
            pl.BlockSpec((_TB, h), lambda i: (i, 0)),
            pl.BlockSpec((e, h), lambda i: (0, 0)),
        ],
        out_specs=[
            pl.BlockSpec((_TB, _TOP_K), lambda i: (i, 0)),
            pl.BlockSpec((_TB, _TOP_K), lambda i: (i, 0)),
            pl.BlockSpec((8, e), lambda i: (0, 0)),
        ],
        out_shape=[
            jax.ShapeDtypeStruct((t, _TOP_K), jnp.float32),
            jax.ShapeDtypeStruct((t, _TOP_K), jnp.int32),
            jax.ShapeDtypeStruct((8, e), jnp.float32),
        ],
    )(x, gate_weight)

    return (
        wout.reshape(b, s, _TOP_K),
        iout.reshape(b, s, _TOP_K).astype(jnp.int64),
        stats[2, 0],
    )
